# Initial kernel scaffold; baseline (speedup 1.0000x reference)
import jax
import jax.numpy as jnp
from jax import lax
from jax.experimental import pallas as pl
from jax.experimental.pallas import tpu as pltpu
from jax.experimental.pallas import tpu_sc as plsc

N_UNIQUE = 1000000
DIM_EMB = 16
BATCH = 16384
N_FIELDS = 26

R = BATCH * N_FIELDS
NW = 32
RW = R // NW
CHUNK = 128
NCHUNK = RW // CHUNK


def _gather_body(table_hbm, ids_hbm, out_hbm, idx_v, rows_v, sem):
    nc = 2
    wid = lax.axis_index("s") * nc + lax.axis_index("c")
    base = wid * RW
    pltpu.sync_copy(ids_hbm.at[pl.ds(base, RW)], idx_v)

    def step(j, _):
        off = pl.multiple_of(j * CHUNK, 8)
        pltpu.async_copy(
            table_hbm.at[idx_v.at[pl.ds(off, CHUNK)]], rows_v, sem
        ).wait()
        pltpu.sync_copy(rows_v, out_hbm.at[pl.ds(base + off, CHUNK)])
        return 0

    lax.fori_loop(0, NCHUNK, step, 0)


@jax.jit
def _encode(emb_weight, flat_ids):
    mesh = plsc.VectorSubcoreMesh(core_axis_name="c", subcore_axis_name="s")
    fn = pl.kernel(
        _gather_body,
        out_type=jax.ShapeDtypeStruct((R, DIM_EMB), jnp.float32),
        mesh=mesh,
        scratch_types=[
            pltpu.VMEM((RW,), jnp.int32),
            pltpu.VMEM((CHUNK, DIM_EMB), jnp.float32),
            pltpu.SemaphoreType.DMA,
        ],
    )
    return fn(emb_weight, flat_ids)


def kernel(state_ids, emb_weight):
    flat_ids = state_ids.reshape(-1)
    out = _encode(emb_weight, flat_ids)
    return out.reshape(BATCH, N_FIELDS * DIM_EMB)


# speedup vs baseline: 1.0478x; 1.0478x over previous
import jax
import jax.numpy as jnp
from jax import lax
from jax.experimental import pallas as pl
from jax.experimental.pallas import tpu as pltpu
from jax.experimental.pallas import tpu_sc as plsc

N_UNIQUE = 1000000
DIM_EMB = 16
BATCH = 16384
N_FIELDS = 26

R = BATCH * N_FIELDS
NW = 32
RW = R // NW
CHUNK = 128
NCHUNK = RW // CHUNK


def _gather_body(table_hbm, ids_hbm, out_hbm, idx_v, rows_v, sem):
    nc = 2
    wid = lax.axis_index("s") * nc + lax.axis_index("c")
    base = wid * RW
    pltpu.sync_copy(ids_hbm.at[pl.ds(base, RW)], idx_v)

    def step(j, _):
        off = pl.multiple_of(j * CHUNK, 8)
        pltpu.async_copy(
            table_hbm.at[idx_v.at[pl.ds(off, CHUNK)]], rows_v, sem
        ).wait()
        pltpu.sync_copy(rows_v, out_hbm.at[pl.ds(base + off, CHUNK)])
        return 0

    lax.fori_loop(0, NCHUNK, step, 0)


@jax.jit
def _encode(emb_weight, flat_ids):
    mesh = plsc.VectorSubcoreMesh(core_axis_name="c", subcore_axis_name="s")
    fn = pl.kernel(
        _gather_body,
        out_type=jax.ShapeDtypeStruct((R, DIM_EMB), jnp.float32),
        mesh=mesh,
        scratch_types=[
            pltpu.VMEM((RW,), jnp.int32),
            pltpu.VMEM((CHUNK, DIM_EMB), jnp.float32),
            pltpu.SemaphoreType.DMA,
        ],
        compiler_params=pltpu.CompilerParams(use_tc_tiling_on_sc=False),
    )
    return fn(emb_weight, flat_ids)


def kernel(state_ids, emb_weight):
    flat_ids = state_ids.reshape(-1)
    out = _encode(emb_weight, flat_ids)
    return out.reshape(BATCH, N_FIELDS * DIM_EMB)


# speedup vs baseline: 1.1884x; 1.1342x over previous
import jax
import jax.numpy as jnp
from jax import lax
from jax.experimental import pallas as pl
from jax.experimental.pallas import tpu as pltpu
from jax.experimental.pallas import tpu_sc as plsc

N_UNIQUE = 1000000
DIM_EMB = 16
BATCH = 16384
N_FIELDS = 26

R = BATCH * N_FIELDS
NW = 32
RW = R // NW
CHUNK = 128
K = 13
SUPER = K * CHUNK
NSUPER = RW // SUPER


def _gather_body(table_hbm, ids_hbm, out_hbm, idx_v, rows_v, sem0, sem1):
    nc = 2
    wid = lax.axis_index("s") * nc + lax.axis_index("c")
    base = wid * RW
    pltpu.sync_copy(ids_hbm.at[pl.ds(base, RW)], idx_v)

    def fire(s, b, sem):
        soff = pl.multiple_of(s * SUPER, 8)
        for c in range(K):
            pltpu.async_copy(
                table_hbm.at[idx_v.at[pl.ds(soff + c * CHUNK, CHUNK)]],
                rows_v.at[b, pl.ds(c * CHUNK, CHUNK)],
                sem,
            )

    def drain(b, sem):
        for c in range(K):
            pltpu.make_async_copy(
                table_hbm.at[idx_v.at[pl.ds(c * CHUNK, CHUNK)]],
                rows_v.at[b, pl.ds(c * CHUNK, CHUNK)],
                sem,
            ).wait()

    def flush(s, b):
        ooff = pl.multiple_of(base + s * SUPER, 8)
        pltpu.sync_copy(rows_v.at[b], out_hbm.at[pl.ds(ooff, SUPER)])

    fire(0, 0, sem0)

    def pair(p, _):
        s0 = p * 2
        fire(s0 + 1, 1, sem1)
        drain(0, sem0)
        flush(s0, 0)

        @pl.when(s0 + 2 < NSUPER)
        def _():
            fire(s0 + 2, 0, sem0)

        drain(1, sem1)
        flush(s0 + 1, 1)
        return 0

    lax.fori_loop(0, NSUPER // 2, pair, 0)


@jax.jit
def _encode(emb_weight, flat_ids):
    mesh = plsc.VectorSubcoreMesh(core_axis_name="c", subcore_axis_name="s")
    fn = pl.kernel(
        _gather_body,
        out_type=jax.ShapeDtypeStruct((R, DIM_EMB), jnp.float32),
        mesh=mesh,
        scratch_types=[
            pltpu.VMEM((RW,), jnp.int32),
            pltpu.VMEM((2, SUPER, DIM_EMB), jnp.float32),
            pltpu.SemaphoreType.DMA,
            pltpu.SemaphoreType.DMA,
        ],
        compiler_params=pltpu.CompilerParams(use_tc_tiling_on_sc=False),
    )
    return fn(emb_weight, flat_ids)


def kernel(state_ids, emb_weight):
    flat_ids = state_ids.reshape(-1)
    out = _encode(emb_weight, flat_ids)
    return out.reshape(BATCH, N_FIELDS * DIM_EMB)
